# Initial kernel scaffold; baseline (speedup 1.0000x reference)
import jax
import jax.numpy as jnp
from jax.experimental import pallas as pl


def kernel(x, pos, seg, word_emb, pos_emb, seg_emb):
    raise NotImplementedError("write your pallas kernel here")


# speedup vs baseline: 4.0003x; 4.0003x over previous
import math
import jax
import jax.numpy as jnp
from jax.experimental import pallas as pl
from jax.experimental.pallas import tpu as pltpu
from jax.experimental.pallas import tpu_sc as plsc

D_MODEL = 128
SCALE = math.sqrt(D_MODEL)
WINDOW = 128


def _seg_embedding_sc(xi, pi, si, word_emb, pos_emb, seg_emb):
    n_tok = xi.shape[1]
    mesh = plsc.VectorSubcoreMesh(core_axis_name="core",
                                  subcore_axis_name="subcore")

    @pl.kernel(
        out_type=jax.ShapeDtypeStruct((n_tok, D_MODEL), jnp.float32),
        mesh=mesh,
        scratch_types=[
            pltpu.VMEM((WINDOW, D_MODEL), jnp.float32),
            pltpu.VMEM((WINDOW, D_MODEL), jnp.float32),
            pltpu.VMEM((WINDOW, D_MODEL), jnp.float32),
            pltpu.SemaphoreType.DMA,
        ],
    )
    def kern(word_hbm, pos_hbm, seg_hbm, xi_hbm, pi_hbm, si_hbm, o_hbm,
             w_v, p_v, s_v, sem):
        def body(xi_v, pi_v, si_v, o_v):
            cw = pltpu.async_copy(word_hbm.at[xi_v.at[0]], w_v, sem)
            cp = pltpu.async_copy(pos_hbm.at[pi_v.at[0]], p_v, sem)
            cs = pltpu.async_copy(seg_hbm.at[si_v.at[0]], s_v, sem)
            cw.wait()
            cp.wait()
            cs.wait()

            @pl.loop(0, WINDOW)
            def _(r):
                @pl.loop(0, D_MODEL, step=16)
                def _(c):
                    sl = (r, pl.ds(c, 16))
                    o_v[sl] = w_v[sl] * SCALE + p_v[sl] + s_v[sl]

        pltpu.emit_pipeline(
            body,
            grid=(n_tok // WINDOW,),
            in_specs=[
                pl.BlockSpec((1, WINDOW), index_map=lambda i: (0, i)),
                pl.BlockSpec((1, WINDOW), index_map=lambda i: (0, i)),
                pl.BlockSpec((1, WINDOW), index_map=lambda i: (0, i)),
            ],
            out_specs=[
                pl.BlockSpec((WINDOW, D_MODEL), index_map=lambda i: (i, 0)),
            ],
            core_axis_name=("core", "subcore"),
            dimension_semantics=(pltpu.PARALLEL,),
        )(xi_hbm, pi_hbm, si_hbm, o_hbm)

    return kern(word_emb, pos_emb, seg_emb, xi, pi, si)


def kernel(x, pos, seg, word_emb, pos_emb, seg_emb):
    b, l = x.shape
    n_tok = b * l
    xi = x.reshape(1, n_tok).astype(jnp.int32)
    pi = pos.reshape(1, n_tok).astype(jnp.int32)
    si = seg.reshape(1, n_tok).astype(jnp.int32)
    out = _seg_embedding_sc(xi, pi, si, word_emb, pos_emb, seg_emb)
    return out.reshape(b, l, D_MODEL)


# speedup vs baseline: 4.2011x; 1.0502x over previous
import math
import jax
import jax.numpy as jnp
from jax.experimental import pallas as pl
from jax.experimental.pallas import tpu as pltpu
from jax.experimental.pallas import tpu_sc as plsc

D_MODEL = 128
SCALE = math.sqrt(D_MODEL)
WINDOW = 128


def _seg_embedding_sc(xi, pi, si, word_emb, pos_emb, seg_emb):
    n_tok = xi.shape[1]
    mesh = plsc.VectorSubcoreMesh(core_axis_name="core",
                                  subcore_axis_name="subcore")

    @pl.kernel(
        out_type=jax.ShapeDtypeStruct((n_tok, D_MODEL), jnp.float32),
        mesh=mesh,
        scratch_types=[
            pltpu.VMEM((WINDOW, D_MODEL), jnp.float32),
            pltpu.VMEM((WINDOW, D_MODEL), jnp.float32),
            pltpu.VMEM((WINDOW, D_MODEL), jnp.float32),
            pltpu.SemaphoreType.DMA,
        ],
    )
    def kern(word_hbm, pos_hbm, seg_hbm, xi_hbm, pi_hbm, si_hbm, o_hbm,
             w_v, p_v, s_v, sem):
        def body(xi_v, pi_v, si_v, o_v):
            cw = pltpu.async_copy(word_hbm.at[xi_v.at[0]], w_v, sem)
            cp = pltpu.async_copy(pos_hbm.at[pi_v.at[0]], p_v, sem)
            cw.wait()
            cp.wait()
            cs = pltpu.async_copy(seg_hbm.at[si_v.at[0]], p_v, sem, add=True)
            cs.wait()

            @pl.loop(0, WINDOW)
            def _(r):
                for c in range(0, D_MODEL, 16):
                    sl = (r, pl.ds(c, 16))
                    o_v[sl] = w_v[sl] * SCALE + p_v[sl]

        pltpu.emit_pipeline(
            body,
            grid=(n_tok // WINDOW,),
            in_specs=[
                pl.BlockSpec((1, WINDOW), index_map=lambda i: (0, i)),
                pl.BlockSpec((1, WINDOW), index_map=lambda i: (0, i)),
                pl.BlockSpec((1, WINDOW), index_map=lambda i: (0, i)),
            ],
            out_specs=[
                pl.BlockSpec((WINDOW, D_MODEL), index_map=lambda i: (i, 0)),
            ],
            core_axis_name=("core", "subcore"),
            dimension_semantics=(pltpu.PARALLEL,),
        )(xi_hbm, pi_hbm, si_hbm, o_hbm)

    return kern(word_emb, pos_emb, seg_emb, xi, pi, si)


def kernel(x, pos, seg, word_emb, pos_emb, seg_emb):
    b, l = x.shape
    n_tok = b * l
    xi = x.reshape(1, n_tok).astype(jnp.int32)
    pi = pos.reshape(1, n_tok).astype(jnp.int32)
    si = seg.reshape(1, n_tok).astype(jnp.int32)
    out = _seg_embedding_sc(xi, pi, si, word_emb, pos_emb, seg_emb)
    return out.reshape(b, l, D_MODEL)


# speedup vs baseline: 9.2001x; 2.1899x over previous
import math
import jax
import jax.numpy as jnp
from jax import lax
from jax.experimental import pallas as pl
from jax.experimental.pallas import tpu as pltpu
from jax.experimental.pallas import tpu_sc as plsc

D = 128
SCALE = math.sqrt(D)
W = 128
N_WORKERS = 32


def _seg_embedding_sc(xi, pi, si, word_emb, pos_emb, seg_emb):
    n_tok = xi.shape[0]
    per_w = n_tok // N_WORKERS
    n_chunks = per_w // W
    mesh = plsc.VectorSubcoreMesh(core_axis_name="core",
                                  subcore_axis_name="subcore")

    @pl.kernel(
        out_type=jax.ShapeDtypeStruct((n_tok, D), jnp.float32),
        mesh=mesh,
        scratch_types=[
            pltpu.VMEM((per_w,), jnp.int32),
            pltpu.VMEM((per_w,), jnp.int32),
            pltpu.VMEM((per_w,), jnp.int32),
            pltpu.VMEM((W, D), jnp.float32),
            pltpu.VMEM((W, D), jnp.float32),
            pltpu.VMEM((W, D), jnp.float32),
            pltpu.VMEM((W, D), jnp.float32),
            pltpu.VMEM((W, D), jnp.float32),
            pltpu.VMEM((W, D), jnp.float32),
            pltpu.SemaphoreType.DMA,
            pltpu.SemaphoreType.DMA,
            pltpu.SemaphoreType.DMA,
            pltpu.SemaphoreType.DMA,
            pltpu.SemaphoreType.DMA,
            pltpu.SemaphoreType.DMA,
            pltpu.SemaphoreType.DMA,
            pltpu.SemaphoreType.DMA,
        ],
    )
    def kern(word_hbm, pos_hbm, seg_hbm, xi_hbm, pi_hbm, si_hbm, o_hbm,
             xv, pv, sv, w0, w1, ps0, ps1, o0, o1,
             sw0, sw1, sp0, sp1, ss0, ss1, so0, so1):
        wid = lax.axis_index("core") * 16 + lax.axis_index("subcore")
        base = wid * per_w
        wbuf = (w0, w1)
        psbuf = (ps0, ps1)
        obuf = (o0, o1)
        sw = (sw0, sw1)
        sp = (sp0, sp1)
        ss = (ss0, ss1)
        so = (so0, so1)

        cx = pltpu.async_copy(xi_hbm.at[pl.ds(base, per_w)], xv, sw0)
        cp_ = pltpu.async_copy(pi_hbm.at[pl.ds(base, per_w)], pv, sp0)
        cs_ = pltpu.async_copy(si_hbm.at[pl.ds(base, per_w)], sv, ss0)
        cx.wait()
        cp_.wait()
        cs_.wait()

        def issue_w(c, q):
            pltpu.async_copy(word_hbm.at[xv.at[pl.ds(c * W, W)]],
                             wbuf[q], sw[q])

        def issue_p(c, q):
            pltpu.async_copy(pos_hbm.at[pv.at[pl.ds(c * W, W)]],
                             psbuf[q], sp[q])

        def issue_s(c, q):
            pltpu.async_copy(seg_hbm.at[sv.at[pl.ds(c * W, W)]],
                             psbuf[q], ss[q], add=True)

        def wait(sem, buf):
            pltpu.make_async_copy(o_hbm.at[pl.ds(0, W)], buf, sem).wait()

        issue_w(0, 0)
        issue_p(0, 0)
        wait(sp[0], psbuf[0])
        issue_s(0, 0)

        def body(c, q):
            @pl.when(c >= 2)
            def _():
                pltpu.make_async_copy(
                    obuf[q], o_hbm.at[pl.ds(base, W)], so[q]).wait()

            @pl.when(c + 1 < n_chunks)
            def _():
                issue_w(c + 1, 1 - q)
                issue_p(c + 1, 1 - q)

            wait(sw[q], wbuf[q])
            wait(ss[q], psbuf[q])

            @pl.loop(0, W, step=2)
            def _(r):
                for rr in range(2):
                    for col in range(0, D, 16):
                        sl = (r + rr, pl.ds(col, 16))
                        obuf[q][sl] = wbuf[q][sl] * SCALE + psbuf[q][sl]

            @pl.when(c + 1 < n_chunks)
            def _():
                wait(sp[1 - q], psbuf[1 - q])
                issue_s(c + 1, 1 - q)

            pltpu.async_copy(obuf[q], o_hbm.at[pl.ds(base + c * W, W)], so[q])

        @pl.loop(0, n_chunks, step=2)
        def _(c):
            body(c, 0)
            body(c + 1, 1)

        pltpu.make_async_copy(obuf[0], o_hbm.at[pl.ds(base, W)], so[0]).wait()
        pltpu.make_async_copy(obuf[1], o_hbm.at[pl.ds(base, W)], so[1]).wait()

    return kern(word_emb, pos_emb, seg_emb, xi, pi, si)


def kernel(x, pos, seg, word_emb, pos_emb, seg_emb):
    b, l = x.shape
    n_tok = b * l
    xi = x.reshape(n_tok).astype(jnp.int32)
    pi = pos.reshape(n_tok).astype(jnp.int32)
    si = seg.reshape(n_tok).astype(jnp.int32)
    out = _seg_embedding_sc(xi, pi, si, word_emb, pos_emb, seg_emb)
    return out.reshape(b, l, D)
